# baseline (speedup 1.0000x reference)
import functools

import jax
import jax.numpy as jnp
from jax import lax
from jax.experimental import pallas as pl

NUM_Q = 4
CB_DIM = 64
CB_SIZE = 1024
BETA = 0.25
B, H, W = 8, 32, 32
T = H * W
N = B * T


def _vq_kernel(x_ref, cb_ref, cbt_ref, quant_ref, idx_ref, loss_ref, perp_ref):
    cb = cb_ref[0]
    cbt = cbt_ref[0]
    cb_norm = jnp.sum(cb * cb, axis=1, keepdims=True)

    iota_f = lax.broadcasted_iota(jnp.int32, (CB_SIZE, T), 0).astype(jnp.float32)
    ones_t = jnp.ones((T,), dtype=jnp.float32)

    loss_acc = jnp.float32(0.0)
    counts = jnp.zeros((CB_SIZE,), dtype=jnp.float32)
    for b in range(B):
        xb = x_ref[b].reshape(CB_DIM, T)
        xnorm = jnp.sum(xb * xb, axis=0, keepdims=True)
        prod = jnp.dot(cb, xb, preferred_element_type=jnp.float32)
        dist = (xnorm - 2.0 * prod) + cb_norm
        minv = jnp.min(dist, axis=0, keepdims=True)
        masked = jnp.where(dist <= minv, iota_f, jnp.float32(CB_SIZE))
        idxf = jnp.min(masked, axis=0)
        idx_ref[0, 0, b, :] = idxf.astype(jnp.int32)
        onehot = (iota_f == idxf[None, :]).astype(jnp.float32)
        quant = jnp.dot(cbt, onehot, preferred_element_type=jnp.float32)
        quant_ref[b] = quant.reshape(CB_DIM, H, W)
        loss_acc += jnp.sum(minv)
        counts += jnp.dot(onehot, ones_t, preferred_element_type=jnp.float32)

    avg = counts / jnp.float32(N)
    perp = jnp.exp(-jnp.sum(avg * jnp.log(avg + 1e-10)))
    loss = (1.0 + BETA) * loss_acc / jnp.float32(N * CB_DIM)
    loss_ref[0, 0, :] = jnp.full((128,), loss, dtype=jnp.float32)
    perp_ref[0, 0, :] = jnp.full((128,), perp, dtype=jnp.float32)


@functools.partial(jax.jit, static_argnames=())
def kernel(x, codebooks):
    cbt = jnp.transpose(codebooks, (0, 2, 1))

    grid = (NUM_Q,)
    quant, idx, loss, perp = pl.pallas_call(
        _vq_kernel,
        grid=grid,
        in_specs=[
            pl.BlockSpec((B, CB_DIM, H, W), lambda i: (0, i, 0, 0)),
            pl.BlockSpec((1, CB_SIZE, CB_DIM), lambda i: (i, 0, 0)),
            pl.BlockSpec((1, CB_DIM, CB_SIZE), lambda i: (i, 0, 0)),
        ],
        out_specs=[
            pl.BlockSpec((B, CB_DIM, H, W), lambda i: (0, i, 0, 0)),
            pl.BlockSpec((1, 1, B, T), lambda i: (i, 0, 0, 0)),
            pl.BlockSpec((1, 1, 128), lambda i: (i, 0, 0)),
            pl.BlockSpec((1, 1, 128), lambda i: (i, 0, 0)),
        ],
        out_shape=[
            jax.ShapeDtypeStruct((B, NUM_Q * CB_DIM, H, W), jnp.float32),
            jax.ShapeDtypeStruct((NUM_Q, 1, B, T), jnp.int32),
            jax.ShapeDtypeStruct((NUM_Q, 1, 128), jnp.float32),
            jax.ShapeDtypeStruct((NUM_Q, 1, 128), jnp.float32),
        ],
    )(x, codebooks, cbt)

    indices_cat = jnp.transpose(idx[:, 0], (1, 0, 2)).reshape(B, NUM_Q, H, W)
    loss_cat = loss[:, 0, 0]
    perplexity_cat = perp[:, 0, 0]
    return (quant, indices_cat, loss_cat, perplexity_cat)


# speedup vs baseline: 1.1106x; 1.1106x over previous
import functools

import jax
import jax.numpy as jnp
from jax import lax
from jax.experimental import pallas as pl

NUM_Q = 4
CB_DIM = 64
CB_SIZE = 1024
BETA = 0.25
B, H, W = 8, 32, 32
T = H * W
N = B * T


def _vq_kernel(x_ref, cb_ref, cbt_ref, quant_ref, idx_ref, loss_ref, perp_ref):
    cb2 = cb_ref[0]
    cbt = cbt_ref[0]
    cb_norm = 0.25 * jnp.sum(cb2 * cb2, axis=1, keepdims=True)

    iota_i = lax.broadcasted_iota(jnp.int32, (CB_SIZE, T), 0)
    ones_t = jnp.ones((T,), dtype=jnp.float32)

    loss_acc = jnp.float32(0.0)
    counts = jnp.zeros((CB_SIZE,), dtype=jnp.float32)
    for b in range(B):
        xb = x_ref[b, 0]
        xnorm = jnp.sum(xb * xb, axis=0, keepdims=True)
        prod2 = jnp.dot(cb2, xb, preferred_element_type=jnp.float32)
        dist = (xnorm + prod2) + cb_norm
        idx = jnp.argmin(dist, axis=0)
        idx_ref[0, 0, b, :] = idx
        onehot = (iota_i == idx[None, :]).astype(jnp.float32)
        quant = jnp.dot(cbt, onehot, preferred_element_type=jnp.float32)
        quant_ref[b, 0] = quant
        diff = quant - xb
        loss_acc += jnp.sum(diff * diff)
        counts += jnp.dot(onehot, ones_t, preferred_element_type=jnp.float32)

    avg = counts / jnp.float32(N)
    perp = jnp.exp(-jnp.sum(avg * jnp.log(avg + 1e-10)))
    loss = (1.0 + BETA) * loss_acc / jnp.float32(N * CB_DIM)
    loss_ref[0, 0, :] = jnp.full((128,), loss, dtype=jnp.float32)
    perp_ref[0, 0, :] = jnp.full((128,), perp, dtype=jnp.float32)


@functools.partial(jax.jit, static_argnames=())
def kernel(x, codebooks):
    x4 = x.reshape(B, NUM_Q, CB_DIM, T)
    cb2 = -2.0 * codebooks
    cbt = jnp.transpose(codebooks, (0, 2, 1))

    grid = (NUM_Q,)
    quant, idx, loss, perp = pl.pallas_call(
        _vq_kernel,
        grid=grid,
        in_specs=[
            pl.BlockSpec((B, 1, CB_DIM, T), lambda i: (0, i, 0, 0)),
            pl.BlockSpec((1, CB_SIZE, CB_DIM), lambda i: (i, 0, 0)),
            pl.BlockSpec((1, CB_DIM, CB_SIZE), lambda i: (i, 0, 0)),
        ],
        out_specs=[
            pl.BlockSpec((B, 1, CB_DIM, T), lambda i: (0, i, 0, 0)),
            pl.BlockSpec((1, 1, B, T), lambda i: (i, 0, 0, 0)),
            pl.BlockSpec((1, 1, 128), lambda i: (i, 0, 0)),
            pl.BlockSpec((1, 1, 128), lambda i: (i, 0, 0)),
        ],
        out_shape=[
            jax.ShapeDtypeStruct((B, NUM_Q, CB_DIM, T), jnp.float32),
            jax.ShapeDtypeStruct((NUM_Q, 1, B, T), jnp.int32),
            jax.ShapeDtypeStruct((NUM_Q, 1, 128), jnp.float32),
            jax.ShapeDtypeStruct((NUM_Q, 1, 128), jnp.float32),
        ],
    )(x4, cb2, cbt)

    quantized_cat = quant.reshape(B, NUM_Q * CB_DIM, H, W)
    indices_cat = jnp.transpose(idx[:, 0], (1, 0, 2)).reshape(B, NUM_Q, H, W)
    loss_cat = loss[:, 0, 0]
    perplexity_cat = perp[:, 0, 0]
    return (quantized_cat, indices_cat, loss_cat, perplexity_cat)


# speedup vs baseline: 1.1168x; 1.0056x over previous
import functools

import jax
import jax.numpy as jnp
from jax import lax
from jax.experimental import pallas as pl

NUM_Q = 4
CB_DIM = 64
CB_SIZE = 1024
BETA = 0.25
B, H, W = 8, 32, 32
T = H * W
N = B * T


def _vq_kernel(x_ref, cb_ref, cbt_ref, quant_ref, idx_ref, loss_ref, perp_ref):
    cb2 = cb_ref[0]
    cbt = cbt_ref[0]
    cb_norm = 0.25 * jnp.sum(cb2 * cb2, axis=1, keepdims=True)

    iota_i = lax.broadcasted_iota(jnp.int32, (CB_SIZE, T), 0)

    loss_acc = jnp.float32(0.0)
    counts = jnp.zeros((CB_SIZE, 1), dtype=jnp.float32)
    for b in range(B):
        xb = x_ref[b, 0]
        xnorm = jnp.sum(xb * xb, axis=0, keepdims=True)
        prod2 = jnp.dot(cb2, xb, preferred_element_type=jnp.float32)
        dist = (xnorm + prod2) + cb_norm
        idx = jnp.argmin(dist, axis=0)
        idx_ref[0, 0, b, :] = idx
        onehot = (iota_i == idx[None, :]).astype(jnp.float32)
        quant = jnp.dot(cbt, onehot, preferred_element_type=jnp.float32)
        quant_ref[b, 0] = quant
        diff = quant - xb
        loss_acc += jnp.sum(diff * diff)
        counts += jnp.sum(onehot, axis=1, keepdims=True)

    avg = counts / jnp.float32(N)
    perp = jnp.exp(-jnp.sum(avg * jnp.log(avg + 1e-10)))
    loss = (1.0 + BETA) * loss_acc / jnp.float32(N * CB_DIM)
    loss_ref[0, 0, :] = jnp.full((128,), loss, dtype=jnp.float32)
    perp_ref[0, 0, :] = jnp.full((128,), perp, dtype=jnp.float32)


@functools.partial(jax.jit, static_argnames=())
def kernel(x, codebooks):
    x4 = x.reshape(B, NUM_Q, CB_DIM, T)
    cb2 = -2.0 * codebooks
    cbt = jnp.transpose(codebooks, (0, 2, 1))

    grid = (NUM_Q,)
    quant, idx, loss, perp = pl.pallas_call(
        _vq_kernel,
        grid=grid,
        in_specs=[
            pl.BlockSpec((B, 1, CB_DIM, T), lambda i: (0, i, 0, 0)),
            pl.BlockSpec((1, CB_SIZE, CB_DIM), lambda i: (i, 0, 0)),
            pl.BlockSpec((1, CB_DIM, CB_SIZE), lambda i: (i, 0, 0)),
        ],
        out_specs=[
            pl.BlockSpec((B, 1, CB_DIM, T), lambda i: (0, i, 0, 0)),
            pl.BlockSpec((1, 1, B, T), lambda i: (i, 0, 0, 0)),
            pl.BlockSpec((1, 1, 128), lambda i: (i, 0, 0)),
            pl.BlockSpec((1, 1, 128), lambda i: (i, 0, 0)),
        ],
        out_shape=[
            jax.ShapeDtypeStruct((B, NUM_Q, CB_DIM, T), jnp.float32),
            jax.ShapeDtypeStruct((NUM_Q, 1, B, T), jnp.int32),
            jax.ShapeDtypeStruct((NUM_Q, 1, 128), jnp.float32),
            jax.ShapeDtypeStruct((NUM_Q, 1, 128), jnp.float32),
        ],
    )(x4, cb2, cbt)

    quantized_cat = quant.reshape(B, NUM_Q * CB_DIM, H, W)
    indices_cat = jnp.transpose(idx[:, 0], (1, 0, 2)).reshape(B, NUM_Q, H, W)
    loss_cat = loss[:, 0, 0]
    perplexity_cat = perp[:, 0, 0]
    return (quantized_cat, indices_cat, loss_cat, perplexity_cat)
